# Initial kernel scaffold; baseline (speedup 1.0000x reference)
import jax
import jax.numpy as jnp
from jax.experimental import pallas as pl


def kernel(mol_embeddings, concentrations, roles, batch_indices, W1, b1, W2, b2, W3, b3):
    raise NotImplementedError("write your pallas kernel here")


# speedup vs baseline: 1.9386x; 1.9386x over previous
import functools

import jax
import jax.numpy as jnp
from jax import lax
from jax.experimental import pallas as pl
from jax.experimental.pallas import tpu as pltpu

_NUM_SEGMENTS = 10000
_WINDOW = 128


def _pick_block(n):
    for b in (640, 512, 320, 256, 128, 64, 32, 16, 8):
        if n % b == 0:
            return b
    return 1


def _body(emb_ref, conc_ref, roles_ref, idx_ref, w1e_ref, wcr_ref, b1_ref,
          w2_ref, b2_ref, w3_ref, b3_ref, out_ref, acc_ref, *, nb, bsz, s):
    i = pl.program_id(0)
    w = _WINDOW

    @pl.when(i == 0)
    def _init():
        acc_ref[...] = jnp.zeros_like(acc_ref)

    h = lax.dot_general(emb_ref[...], w1e_ref[...], (((1,), (0,)), ((), ())),
                        preferred_element_type=jnp.float32)
    h = h + conc_ref[...] * wcr_ref[0:1, :] + roles_ref[...] * wcr_ref[1:2, :]
    h = jnp.maximum(h + b1_ref[...], 0.0)

    idx = idx_ref[...].reshape(1, bsz)
    pos = lax.broadcasted_iota(jnp.int32, (1, bsz), 1)

    def cond(p):
        return p < bsz

    def step(p):
        base = jnp.min(jnp.where(pos >= p, idx, s))
        base = (base // 8) * 8
        local = jnp.broadcast_to(idx - base, (w, bsz))
        row = lax.broadcasted_iota(jnp.int32, (w, bsz), 0)
        live = jnp.broadcast_to(pos >= p, (w, bsz))
        oh = ((local == row) & live).astype(jnp.float32)
        part = lax.dot_general(oh, h, (((1,), (0,)), ((), ())),
                               preferred_element_type=jnp.float32)
        acc_ref[pl.ds(base, w), :] = acc_ref[pl.ds(base, w), :] + part
        return p + jnp.sum(oh).astype(jnp.int32)

    lax.while_loop(cond, step, jnp.int32(0))

    @pl.when(i == nb - 1)
    def _tail():
        pooled = acc_ref[0:s, :]
        x = lax.dot_general(pooled, w2_ref[...], (((1,), (0,)), ((), ())),
                            preferred_element_type=jnp.float32)
        x = jnp.maximum(x + b2_ref[...], 0.0)
        pred = lax.dot_general(x, w3_ref[...], (((1,), (0,)), ((), ())),
                               preferred_element_type=jnp.float32)
        out_ref[...] = pred + b3_ref[...]


def kernel(mol_embeddings, concentrations, roles, batch_indices,
           W1, b1, W2, b2, W3, b3):
    n, d = mol_embeddings.shape
    h_dim = W1.shape[1]
    s = _NUM_SEGMENTS
    bsz = _pick_block(n)
    nb = n // bsz
    sp = ((s + _WINDOW + 7) // 8) * 8

    idx3 = batch_indices.reshape(nb, 1, bsz)
    w1e = W1[:d]
    wcr = W1[d:]
    b1r = b1.reshape(1, h_dim)
    b2r = b2.reshape(1, h_dim)
    b3r = b3.reshape(1, 1)

    out = pl.pallas_call(
        functools.partial(_body, nb=nb, bsz=bsz, s=s),
        grid=(nb,),
        in_specs=[
            pl.BlockSpec((bsz, d), lambda i: (i, 0)),
            pl.BlockSpec((bsz, 1), lambda i: (i, 0)),
            pl.BlockSpec((bsz, 1), lambda i: (i, 0)),
            pl.BlockSpec((1, 1, bsz), lambda i: (i, 0, 0)),
            pl.BlockSpec((d, h_dim), lambda i: (0, 0)),
            pl.BlockSpec((2, h_dim), lambda i: (0, 0)),
            pl.BlockSpec((1, h_dim), lambda i: (0, 0)),
            pl.BlockSpec((h_dim, h_dim), lambda i: (0, 0)),
            pl.BlockSpec((1, h_dim), lambda i: (0, 0)),
            pl.BlockSpec((h_dim, 1), lambda i: (0, 0)),
            pl.BlockSpec((1, 1), lambda i: (0, 0)),
        ],
        out_specs=pl.BlockSpec((s, 1), lambda i: (0, 0)),
        out_shape=jax.ShapeDtypeStruct((s, 1), jnp.float32),
        scratch_shapes=[pltpu.VMEM((sp, h_dim), jnp.float32)],
    )(mol_embeddings, concentrations, roles, idx3, w1e, wcr, b1r,
      W2, b2r, W3, b3r)
    return out[:, 0]


# speedup vs baseline: 2.0181x; 1.0410x over previous
import functools

import jax
import jax.numpy as jnp
from jax import lax
from jax.experimental import pallas as pl
from jax.experimental.pallas import tpu as pltpu

_NUM_SEGMENTS = 10000
_WINDOW = 64


def _pick_block(n):
    for b in (640, 512, 320, 256, 128, 64, 32, 16, 8):
        if n % b == 0:
            return b
    return 1


def _body(emb_ref, conc_ref, roles_ref, idx_ref, w1e_ref, wcr_ref, b1_ref,
          w2_ref, b2_ref, w3_ref, b3_ref, out_ref, acc_ref, *, nb, bsz, s):
    i = pl.program_id(0)
    w = _WINDOW

    @pl.when(i == 0)
    def _init():
        acc_ref[...] = jnp.zeros_like(acc_ref)

    h = lax.dot_general(emb_ref[...].astype(jnp.bfloat16), w1e_ref[...],
                        (((1,), (0,)), ((), ())),
                        preferred_element_type=jnp.float32)
    h = h + conc_ref[...] * wcr_ref[0:1, :] + roles_ref[...] * wcr_ref[1:2, :]
    h = jnp.maximum(h + b1_ref[...], 0.0).astype(jnp.bfloat16)

    idx = idx_ref[...].reshape(1, bsz)
    pos = lax.broadcasted_iota(jnp.int32, (1, bsz), 1)
    row = lax.broadcasted_iota(jnp.int32, (w, bsz), 0)

    def cond(p):
        return p < bsz

    def step(p):
        live1 = pos >= p
        base = jnp.min(jnp.where(live1, idx, s))
        base = (base // 8) * 8
        local1 = idx - base
        local = jnp.broadcast_to(local1, (w, bsz))
        live = jnp.broadcast_to(live1, (w, bsz))
        oh = jnp.where((local == row) & live,
                       jnp.bfloat16(1), jnp.bfloat16(0))
        part = lax.dot_general(oh, h, (((1,), (0,)), ((), ())),
                               preferred_element_type=jnp.float32)
        acc_ref[pl.ds(base, w), :] = acc_ref[pl.ds(base, w), :] + part
        cnt = jnp.sum(((local1 < w) & live1).astype(jnp.int32))
        return p + cnt

    lax.while_loop(cond, step, jnp.int32(0))

    @pl.when(i == nb - 1)
    def _tail():
        pooled = acc_ref[0:s, :]
        x = lax.dot_general(pooled, w2_ref[...], (((1,), (0,)), ((), ())),
                            preferred_element_type=jnp.float32)
        x = jnp.maximum(x + b2_ref[...], 0.0)
        pred = lax.dot_general(x, w3_ref[...], (((1,), (0,)), ((), ())),
                               preferred_element_type=jnp.float32)
        out_ref[...] = pred + b3_ref[...]


def kernel(mol_embeddings, concentrations, roles, batch_indices,
           W1, b1, W2, b2, W3, b3):
    n, d = mol_embeddings.shape
    h_dim = W1.shape[1]
    s = _NUM_SEGMENTS
    bsz = _pick_block(n)
    nb = n // bsz
    sp = ((s + _WINDOW + 7) // 8) * 8

    idx3 = batch_indices.reshape(nb, 1, bsz)
    w1e = W1[:d].astype(jnp.bfloat16)
    wcr = W1[d:]
    b1r = b1.reshape(1, h_dim)
    b2r = b2.reshape(1, h_dim)
    b3r = b3.reshape(1, 1)

    out = pl.pallas_call(
        functools.partial(_body, nb=nb, bsz=bsz, s=s),
        grid=(nb,),
        in_specs=[
            pl.BlockSpec((bsz, d), lambda i: (i, 0)),
            pl.BlockSpec((bsz, 1), lambda i: (i, 0)),
            pl.BlockSpec((bsz, 1), lambda i: (i, 0)),
            pl.BlockSpec((1, 1, bsz), lambda i: (i, 0, 0)),
            pl.BlockSpec((d, h_dim), lambda i: (0, 0)),
            pl.BlockSpec((2, h_dim), lambda i: (0, 0)),
            pl.BlockSpec((1, h_dim), lambda i: (0, 0)),
            pl.BlockSpec((h_dim, h_dim), lambda i: (0, 0)),
            pl.BlockSpec((1, h_dim), lambda i: (0, 0)),
            pl.BlockSpec((h_dim, 1), lambda i: (0, 0)),
            pl.BlockSpec((1, 1), lambda i: (0, 0)),
        ],
        out_specs=pl.BlockSpec((s, 1), lambda i: (0, 0)),
        out_shape=jax.ShapeDtypeStruct((s, 1), jnp.float32),
        scratch_shapes=[pltpu.VMEM((sp, h_dim), jnp.float32)],
    )(mol_embeddings, concentrations, roles, idx3, w1e, wcr, b1r,
      W2, b2r, W3, b3r)
    return out[:, 0]
